# baseline (speedup 1.0000x reference)
import functools

import jax
import jax.numpy as jnp
from jax import lax
from jax.experimental import pallas as pl
from jax.experimental.pallas import tpu as pltpu
from jax.experimental.pallas import tpu_sc as plsc

B = 4096
SEQ = 50
D = 128
NL = 16
NW = 32
ROWS_PER_W = B // NW
ROWS_PER_CHUNK = 2
IDX_PER_CHUNK = ROWS_PER_CHUNK * SEQ
CHUNKS = ROWS_PER_W // ROWS_PER_CHUNK


NBUF = 8


def _sc_pool(idx_hbm, table_hbm, out_hbm, idx_v, rows_v, pooled_v,
             sem0, sem1, sem2, sem3, sem4, sem5, sem6, sem7, osem):
    c = lax.axis_index("c")
    s = lax.axis_index("s")
    wid = s * 2 + c
    sems = (sem0, sem1, sem2, sem3, sem4, sem5, sem6, sem7)
    pltpu.sync_copy(idx_hbm.at[wid], idx_v)

    for b in range(NBUF):
        pltpu.async_copy(table_hbm.at[idx_v.at[b]], rows_v.at[b], sems[b])

    def step_body(g, carry):
        @pl.when(g >= 2)
        def _():
            pltpu.make_async_copy(
                pooled_v.at[0], out_hbm.at[pl.ds(0, NBUF * ROWS_PER_CHUNK)],
                osem).wait()
        for b in range(NBUF):
            j = g * NBUF + b
            pltpu.make_async_copy(
                table_hbm.at[idx_v.at[0]], rows_v.at[b], sems[b]).wait()
            for r in range(ROWS_PER_CHUNK):
                def seq_body(t, accs, r=r, b=b):
                    base = r * SEQ + t * 10
                    out = list(accs)
                    for tt in range(10):
                        for l in range(D // NL):
                            out[l] = out[l] + rows_v[b, base + tt,
                                                     pl.ds(l * NL, NL)]
                    return tuple(out)

                accs = lax.fori_loop(
                    0, SEQ // 10, seq_body,
                    tuple(jnp.zeros((NL,), jnp.float32)
                          for _ in range(D // NL)))
                for l in range(D // NL):
                    pooled_v[g % 2, b * ROWS_PER_CHUNK + r,
                             pl.ds(l * NL, NL)] = (accs[l] * (1.0 / SEQ))
            @pl.when(j + NBUF < CHUNKS)
            def _():
                pltpu.async_copy(
                    table_hbm.at[idx_v.at[j + NBUF]], rows_v.at[b], sems[b])
        out_base = wid * ROWS_PER_W + g * (NBUF * ROWS_PER_CHUNK)
        pltpu.async_copy(
            pooled_v.at[g % 2],
            out_hbm.at[pl.ds(out_base, NBUF * ROWS_PER_CHUNK)], osem)
        return carry

    lax.fori_loop(0, CHUNKS // NBUF, step_body, 0)
    for _ in range(2):
        pltpu.make_async_copy(
            pooled_v.at[0], out_hbm.at[pl.ds(0, NBUF * ROWS_PER_CHUNK)],
            osem).wait()


def _mlp_body(x_ref, w1_ref, b1_ref, w2_ref, b2_ref, w3_ref, b3_ref, o_ref):
    x = x_ref[...]
    h = jnp.maximum(
        jnp.dot(x, w1_ref[...], preferred_element_type=jnp.float32)
        + b1_ref[...], 0.0)
    h = jnp.maximum(
        jnp.dot(h, w2_ref[...], preferred_element_type=jnp.float32)
        + b2_ref[...], 0.0)
    o_ref[...] = (
        jnp.dot(h, w3_ref[...], preferred_element_type=jnp.float32)
        + b3_ref[...])


def kernel(node_sequence, table, W1, b1, W2, b2, W3, b3):
    idx = node_sequence.astype(jnp.int32).reshape(NW, CHUNKS, IDX_PER_CHUNK)

    mesh = plsc.VectorSubcoreMesh(core_axis_name="c", subcore_axis_name="s")
    pooled = pl.kernel(
        _sc_pool,
        mesh=mesh,
        out_type=jax.ShapeDtypeStruct((B, D), jnp.float32),
        scratch_types=[
            pltpu.VMEM((CHUNKS, IDX_PER_CHUNK), jnp.int32),
            pltpu.VMEM((NBUF, IDX_PER_CHUNK, D), jnp.float32),
            pltpu.VMEM((2, NBUF * ROWS_PER_CHUNK, D), jnp.float32),
        ] + [pltpu.SemaphoreType.DMA] * 9,
    )(idx, table)

    bt = 4096
    out = pl.pallas_call(
        _mlp_body,
        grid=(B // bt,),
        in_specs=[
            pl.BlockSpec((bt, D), lambda i: (i, 0)),
            pl.BlockSpec((D, 128), lambda i: (0, 0)),
            pl.BlockSpec((1, 128), lambda i: (0, 0)),
            pl.BlockSpec((128, 64), lambda i: (0, 0)),
            pl.BlockSpec((1, 64), lambda i: (0, 0)),
            pl.BlockSpec((64, 1), lambda i: (0, 0)),
            pl.BlockSpec((1, 1), lambda i: (0, 0)),
        ],
        out_specs=pl.BlockSpec((bt, 1), lambda i: (i, 0)),
        out_shape=jax.ShapeDtypeStruct((B, 1), jnp.float32),
    )(pooled, W1, b1.reshape(1, 128), W2, b2.reshape(1, 64),
      W3, b3.reshape(1, 1))
    return out


# speedup vs baseline: 1.1923x; 1.1923x over previous
import functools

import jax
import jax.numpy as jnp
from jax import lax
from jax.experimental import pallas as pl
from jax.experimental.pallas import tpu as pltpu
from jax.experimental.pallas import tpu_sc as plsc

B = 4096
SEQ = 50
D = 128
NL = 16
NW = 32
ROWS_PER_W = B // NW
ROWS_PER_CHUNK = 2
IDX_PER_CHUNK = ROWS_PER_CHUNK * SEQ
CHUNKS = ROWS_PER_W // ROWS_PER_CHUNK


NBUF = 4


def _sc_pool(idx_hbm, table_hbm, out_hbm, idx_v, rows_v, pooled_v,
             sem0, sem1, sem2, sem3, osem):
    c = lax.axis_index("c")
    s = lax.axis_index("s")
    wid = s * 2 + c
    sems = (sem0, sem1, sem2, sem3)
    pltpu.sync_copy(idx_hbm.at[wid], idx_v)

    for b in range(NBUF):
        pltpu.async_copy(table_hbm.at[idx_v.at[b]], rows_v.at[b], sems[b])

    def step_body(g, carry):
        @pl.when(g >= 2)
        def _():
            pltpu.make_async_copy(
                pooled_v.at[0], out_hbm.at[pl.ds(0, NBUF * ROWS_PER_CHUNK)],
                osem).wait()
        for b in range(NBUF):
            j = g * NBUF + b
            pltpu.make_async_copy(
                table_hbm.at[idx_v.at[0]], rows_v.at[b], sems[b]).wait()
            for r in range(ROWS_PER_CHUNK):
                def seq_body(t, accs, r=r, b=b):
                    base = r * SEQ + t * 10
                    out = list(accs)
                    for tt in range(10):
                        for l in range(D // NL):
                            out[l] = out[l] + rows_v[b, base + tt,
                                                     pl.ds(l * NL, NL)]
                    return tuple(out)

                accs = lax.fori_loop(
                    0, SEQ // 10, seq_body,
                    tuple(jnp.zeros((NL,), jnp.float32)
                          for _ in range(D // NL)))
                for l in range(D // NL):
                    pooled_v[g % 2, b * ROWS_PER_CHUNK + r,
                             pl.ds(l * NL, NL)] = (accs[l] * (1.0 / SEQ))
            @pl.when(j + NBUF < CHUNKS)
            def _():
                pltpu.async_copy(
                    table_hbm.at[idx_v.at[j + NBUF]], rows_v.at[b], sems[b])
        out_base = wid * ROWS_PER_W + g * (NBUF * ROWS_PER_CHUNK)
        pltpu.async_copy(
            pooled_v.at[g % 2],
            out_hbm.at[pl.ds(out_base, NBUF * ROWS_PER_CHUNK)], osem)
        return carry

    lax.fori_loop(0, CHUNKS // NBUF, step_body, 0)
    for _ in range(2):
        pltpu.make_async_copy(
            pooled_v.at[0], out_hbm.at[pl.ds(0, NBUF * ROWS_PER_CHUNK)],
            osem).wait()


def _mlp_body(x_ref, w1_ref, b1_ref, w2_ref, b2_ref, w3_ref, b3_ref, o_ref):
    x = x_ref[...]
    h = jnp.maximum(
        jnp.dot(x, w1_ref[...], preferred_element_type=jnp.float32)
        + b1_ref[...], 0.0)
    h = jnp.maximum(
        jnp.dot(h, w2_ref[...], preferred_element_type=jnp.float32)
        + b2_ref[...], 0.0)
    o_ref[...] = (
        jnp.dot(h, w3_ref[...], preferred_element_type=jnp.float32)
        + b3_ref[...])


def kernel(node_sequence, table, W1, b1, W2, b2, W3, b3):
    idx = node_sequence.astype(jnp.int32).reshape(NW, CHUNKS, IDX_PER_CHUNK)

    mesh = plsc.VectorSubcoreMesh(core_axis_name="c", subcore_axis_name="s")
    pooled = pl.kernel(
        _sc_pool,
        mesh=mesh,
        out_type=jax.ShapeDtypeStruct((B, D), jnp.float32),
        scratch_types=[
            pltpu.VMEM((CHUNKS, IDX_PER_CHUNK), jnp.int32),
            pltpu.VMEM((NBUF, IDX_PER_CHUNK, D), jnp.float32),
            pltpu.VMEM((2, NBUF * ROWS_PER_CHUNK, D), jnp.float32),
        ] + [pltpu.SemaphoreType.DMA] * 5,
    )(idx, table)

    bt = 4096
    out = pl.pallas_call(
        _mlp_body,
        grid=(B // bt,),
        in_specs=[
            pl.BlockSpec((bt, D), lambda i: (i, 0)),
            pl.BlockSpec((D, 128), lambda i: (0, 0)),
            pl.BlockSpec((1, 128), lambda i: (0, 0)),
            pl.BlockSpec((128, 64), lambda i: (0, 0)),
            pl.BlockSpec((1, 64), lambda i: (0, 0)),
            pl.BlockSpec((64, 1), lambda i: (0, 0)),
            pl.BlockSpec((1, 1), lambda i: (0, 0)),
        ],
        out_specs=pl.BlockSpec((bt, 1), lambda i: (i, 0)),
        out_shape=jax.ShapeDtypeStruct((B, 1), jnp.float32),
    )(pooled, W1, b1.reshape(1, 128), W2, b2.reshape(1, 64),
      W3, b3.reshape(1, 1))
    return out
